# speedup vs baseline: 5.9958x; 5.9958x over previous
import functools

import jax
import jax.numpy as jnp
from jax.experimental import pallas as pl
from jax.experimental.pallas import tpu as pltpu

K = 512


def _row_iota(shape):
    return jax.lax.broadcasted_iota(jnp.int32, shape, 0)


def _partner(v, j):
    n, L = v.shape
    r = v.reshape(n // (2 * j), 2 * j, L)
    top = r[:, :j, :]
    bot = r[:, j:, :]
    return jnp.concatenate([bot, top], axis=1).reshape(n, L)


def _cmpex(v, j, take_min):
    pv = _partner(v, j)
    mx = jnp.maximum(v, pv)
    mn = jnp.minimum(v, pv)
    return jnp.where(take_min, mn, mx)


def _bitonic_sort(v, desc):
    n, L = v.shape
    row = _row_iota((n, L))
    k = 2
    while k <= n:
        up = (row & k) == 0
        j = k // 2
        while j >= 1:
            base = ((row & j) == 0) == up
            take_min = jnp.logical_xor(base, desc)
            v = _cmpex(v, j, take_min)
            j //= 2
        k *= 2
    return v


def _bitonic_refine(v, desc, first_j):
    n, L = v.shape
    row = _row_iota((n, L))
    j = first_j
    while j >= 1:
        take_max = (row & j) == 0
        take_min = jnp.logical_xor(take_max, desc)
        v = _cmpex(v, j, take_min)
        j //= 2
    return v


def _topk_kernel(x_ref, o_ref, s_ref, *, n_blocks, n_chan):
    def sort_block(b, _):
        v = x_ref[0, pl.ds(b * K, K), :]
        asc = (b % 2) == 1
        s_ref[pl.ds(b * K, K), :] = _bitonic_sort(v, jnp.logical_not(asc))
        return 0

    jax.lax.fori_loop(0, n_blocks, sort_block, 0)

    n = n_blocks
    while n > 1:
        n //= 2
        last = n == 1

        def merge(i, _, last=last):
            a = s_ref[pl.ds((2 * i) * K, K), :]
            b = s_ref[pl.ds((2 * i + 1) * K, K), :]
            t = jnp.maximum(a, b)
            desc = (i % 2) == 0
            if last:
                lane = jax.lax.broadcasted_iota(jnp.int32, (K, 2 * n_chan), 1)
                desc = jnp.logical_xor(desc, lane >= n_chan)
            s_ref[pl.ds(i * K, K), :] = _bitonic_refine(t, desc, K // 2)
            return 0

        jax.lax.fori_loop(0, n, merge, 0)

    S = s_ref[0:K, :]
    pv = jnp.roll(S, n_chan, axis=1)
    lane = jax.lax.broadcasted_iota(jnp.int32, (K, 2 * n_chan), 1)
    lo = lane < n_chan
    t = jnp.where(lo, jnp.maximum(S, pv), jnp.minimum(S, pv))
    t = _bitonic_refine(t, True, K // 2)
    o_ref[0, :, :] = t[:, :n_chan]


def kernel(x):
    B, L, C = x.shape
    rows = L // 2
    n_blocks = rows // K
    assert rows % K == 0 and n_blocks >= 2 and (n_blocks & (n_blocks - 1)) == 0
    xr = x.reshape(B, rows, 2 * C)

    body = functools.partial(_topk_kernel, n_blocks=n_blocks, n_chan=C)
    out = pl.pallas_call(
        body,
        grid=(B,),
        in_specs=[
            pl.BlockSpec((1, rows, 2 * C), lambda b: (b, 0, 0)),
        ],
        out_specs=pl.BlockSpec((1, K, C), lambda b: (b, 0, 0)),
        out_shape=jax.ShapeDtypeStruct((B, K, C), x.dtype),
        scratch_shapes=[pltpu.VMEM((rows, 2 * C), x.dtype)],
    )(xr)
    return out


# speedup vs baseline: 17.9959x; 3.0014x over previous
import functools

import jax
import jax.numpy as jnp
from jax.experimental import pallas as pl
from jax.experimental.pallas import tpu as pltpu

K = 512
VR = 8
NV = K // VR

def _make_masks(n_chan):
    s = jax.lax.broadcasted_iota(jnp.int32, (VR, 2 * n_chan), 0)
    lane = jax.lax.broadcasted_iota(jnp.int32, (VR, 2 * n_chan), 1)
    mk = {}
    for j in (1, 2, 4):
        mk[('mj', j)] = (s & j) == 0
    for (j, k) in ((1, 2), (2, 4), (1, 4)):
        mk[('mjk', j, k)] = ((s & j) == 0) == ((s & k) == 0)
    lane_ge = lane >= n_chan
    mk['lane_lt'] = lane < n_chan
    for j in (1, 2, 4):
        mk[('mjx', j)] = jnp.logical_xor(mk[('mj', j)], lane_ge)
    return mk


def _sub_partner(mk, v, j):
    if j == 4:
        return pltpu.roll(v, 4, axis=0)
    up = pltpu.roll(v, VR - j, axis=0)
    down = pltpu.roll(v, j, axis=0)
    return jnp.where(mk[('mj', j)], up, down)


def _stage(mk, vs, j, k, desc):
    if j >= VR:
        jv, kv = j // VR, k // VR
        for t in range(NV):
            if t & jv:
                continue
            u = t | jv
            a, b = vs[t], vs[u]
            mx = jnp.maximum(a, b)
            mn = jnp.minimum(a, b)
            if ((t & kv) == 0) != desc:
                vs[t], vs[u] = mn, mx
            else:
                vs[t], vs[u] = mx, mn
    else:
        for t in range(NV):
            pv = _sub_partner(mk, vs[t], j)
            mx = jnp.maximum(vs[t], pv)
            mn = jnp.minimum(vs[t], pv)
            if k >= VR:
                e = ((t & (k // VR)) == 0) != desc
                m = mk[('mj', j)]
                vs[t] = jnp.where(m, mn, mx) if e else jnp.where(m, mx, mn)
            else:
                m = mk[('mjk', j, k)]
                vs[t] = jnp.where(m, mx, mn) if desc else jnp.where(m, mn, mx)


def _sort_block(mk, vs, desc):
    k = 2
    while k <= K:
        j = k // 2
        while j >= 1:
            _stage(mk, vs, j, k, desc)
            j //= 2
        k *= 2


def _refine(mk, vs, desc):
    j = K // 2
    while j >= 1:
        if j >= VR:
            jv = j // VR
            for t in range(NV):
                if t & jv:
                    continue
                u = t | jv
                a, b = vs[t], vs[u]
                mx = jnp.maximum(a, b)
                mn = jnp.minimum(a, b)
                vs[t], vs[u] = (mx, mn) if desc else (mn, mx)
        else:
            for t in range(NV):
                pv = _sub_partner(mk, vs[t], j)
                mx = jnp.maximum(vs[t], pv)
                mn = jnp.minimum(vs[t], pv)
                m = mk[('mj', j)]
                vs[t] = jnp.where(m, mx, mn) if desc else jnp.where(m, mn, mx)
        j //= 2


def _refine_lanesplit(mk, vs):
    lt = mk['lane_lt']
    j = K // 2
    while j >= 1:
        if j >= VR:
            jv = j // VR
            for t in range(NV):
                if t & jv:
                    continue
                u = t | jv
                a, b = vs[t], vs[u]
                mx = jnp.maximum(a, b)
                mn = jnp.minimum(a, b)
                vs[t] = jnp.where(lt, mx, mn)
                vs[u] = jnp.where(lt, mn, mx)
        else:
            for t in range(NV):
                pv = _sub_partner(mk, vs[t], j)
                mx = jnp.maximum(vs[t], pv)
                mn = jnp.minimum(vs[t], pv)
                vs[t] = jnp.where(mk[('mjx', j)], mn, mx)
        j //= 2


def _load_list(ref, base):
    v = ref[pl.ds(base, K), :]
    return [v[VR * i:VR * (i + 1), :] for i in range(NV)]


def _store_list(ref, base, vs):
    ref[pl.ds(base, K), :] = jnp.concatenate(vs, axis=0)


def _merge_pair(mk, s_ref, i, desc, lanesplit=False):
    a = _load_list(s_ref, (2 * i) * K)
    b = _load_list(s_ref, (2 * i + 1) * K)
    t = [jnp.maximum(x, y) for x, y in zip(a, b)]
    if lanesplit:
        _refine_lanesplit(mk, t)
    else:
        _refine(mk, t, desc)
    _store_list(s_ref, i * K, t)


def _topk_kernel(x_ref, o_ref, s_ref, *, n_blocks, n_chan):
    mk = _make_masks(n_chan)

    def sort_pair(p, _):
        for q in (0, 1):
            vs = _load_list(x_ref, (2 * p + q) * K)
            _sort_block(mk, vs, desc=(q == 0))
            _store_list(s_ref, (2 * p + q) * K, vs)
        return 0

    jax.lax.fori_loop(0, n_blocks // 2, sort_pair, 0)

    lists = n_blocks
    while lists > 2:
        m = lists // 2

        def merge_pair2(p, _):
            _merge_pair(mk, s_ref, 2 * p, desc=True)
            _merge_pair(mk, s_ref, 2 * p + 1, desc=False)
            return 0

        jax.lax.fori_loop(0, m // 2, merge_pair2, 0)
        lists = m

    _merge_pair(mk, s_ref, 0, desc=True, lanesplit=True)

    S = _load_list(s_ref, 0)
    lt = mk['lane_lt']
    t = []
    for v in S:
        pv = pltpu.roll(v, n_chan, axis=1)
        t.append(jnp.where(lt, jnp.maximum(v, pv), jnp.minimum(v, pv)))
    _refine(mk, t, desc=True)
    for i in range(NV):
        o_ref[pl.ds(VR * i, VR), :] = t[i][:, :n_chan]


def kernel(x):
    B, L, C = x.shape
    rows = L // 2
    n_blocks = rows // K
    assert rows % K == 0 and n_blocks >= 2 and (n_blocks & (n_blocks - 1)) == 0
    assert C == 64
    xr = x.reshape(B * rows, 2 * C)

    body = functools.partial(_topk_kernel, n_blocks=n_blocks, n_chan=C)
    out = pl.pallas_call(
        body,
        grid=(B,),
        in_specs=[
            pl.BlockSpec((rows, 2 * C), lambda b: (b, 0)),
        ],
        out_specs=pl.BlockSpec((K, C), lambda b: (b, 0)),
        out_shape=jax.ShapeDtypeStruct((B * K, C), x.dtype),
        scratch_shapes=[pltpu.VMEM((rows, 2 * C), x.dtype)],
    )(xr)
    return out.reshape(B, K, C)


# speedup vs baseline: 23.5427x; 1.3082x over previous
import functools

import jax
import jax.numpy as jnp
from jax.experimental import pallas as pl
from jax.experimental.pallas import tpu as pltpu

K = 512
VR = 8
NV = K // VR


def _make_masks(n_chan):
    s = jax.lax.broadcasted_iota(jnp.int32, (VR, 2 * n_chan), 0)
    lane = jax.lax.broadcasted_iota(jnp.int32, (VR, 2 * n_chan), 1)
    mk = {}
    for j in (1, 2, 4):
        mk[('mj', j)] = (s & j) == 0
    for (j, k) in ((1, 2), (2, 4), (1, 4)):
        mk[('mjk', j, k)] = ((s & j) == 0) == ((s & k) == 0)
    lane_ge = lane >= n_chan
    mk['lane_lt'] = lane < n_chan
    for j in (1, 2, 4):
        mk[('mjx', j)] = jnp.logical_xor(mk[('mj', j)], lane_ge)
    return mk


def _sub_partner(mk, v, j):
    if j == 4:
        return pltpu.roll(v, 4, axis=0)
    up = pltpu.roll(v, VR - j, axis=0)
    down = pltpu.roll(v, j, axis=0)
    return jnp.where(mk[('mj', j)], up, down)


def _stage(mk, vs, j, k, desc):
    if j < NV:
        for t in range(NV):
            if t & j:
                continue
            u = t | j
            a, b = vs[t], vs[u]
            mx = jnp.maximum(a, b)
            mn = jnp.minimum(a, b)
            if k <= NV // 2:
                if ((t & k) == 0) != desc:
                    vs[t], vs[u] = mn, mx
                else:
                    vs[t], vs[u] = mx, mn
            elif k == VR * NV:
                if not desc:
                    vs[t], vs[u] = mn, mx
                else:
                    vs[t], vs[u] = mx, mn
            else:
                sm = mk[('mj', k // NV)]
                if not desc:
                    vs[t] = jnp.where(sm, mn, mx)
                    vs[u] = jnp.where(sm, mx, mn)
                else:
                    vs[t] = jnp.where(sm, mx, mn)
                    vs[u] = jnp.where(sm, mn, mx)
    else:
        js = j // NV
        for t in range(NV):
            pv = _sub_partner(mk, vs[t], js)
            mx = jnp.maximum(vs[t], pv)
            mn = jnp.minimum(vs[t], pv)
            if k == VR * NV:
                m = mk[('mj', js)]
            else:
                m = mk[('mjk', js, k // NV)]
            vs[t] = jnp.where(m, mn, mx) if not desc else jnp.where(m, mx, mn)


def _sort_block(mk, vs, desc):
    k = 2
    while k <= K:
        j = k // 2
        while j >= 1:
            _stage(mk, vs, j, k, desc)
            j //= 2
        k *= 2


def _refine(mk, vs, desc):
    j = K // 2
    while j >= 1:
        if j < NV:
            for t in range(NV):
                if t & j:
                    continue
                u = t | j
                a, b = vs[t], vs[u]
                mx = jnp.maximum(a, b)
                mn = jnp.minimum(a, b)
                vs[t], vs[u] = (mx, mn) if desc else (mn, mx)
        else:
            js = j // NV
            for t in range(NV):
                pv = _sub_partner(mk, vs[t], js)
                mx = jnp.maximum(vs[t], pv)
                mn = jnp.minimum(vs[t], pv)
                m = mk[('mj', js)]
                vs[t] = jnp.where(m, mx, mn) if desc else jnp.where(m, mn, mx)
        j //= 2


def _refine_lanesplit(mk, vs):
    lt = mk['lane_lt']
    j = K // 2
    while j >= 1:
        if j < NV:
            for t in range(NV):
                if t & j:
                    continue
                u = t | j
                a, b = vs[t], vs[u]
                mx = jnp.maximum(a, b)
                mn = jnp.minimum(a, b)
                vs[t] = jnp.where(lt, mx, mn)
                vs[u] = jnp.where(lt, mn, mx)
        else:
            js = j // NV
            for t in range(NV):
                pv = _sub_partner(mk, vs[t], js)
                mx = jnp.maximum(vs[t], pv)
                mn = jnp.minimum(vs[t], pv)
                vs[t] = jnp.where(mk[('mjx', js)], mn, mx)
        j //= 2


def _load_list(ref, base):
    v = ref[pl.ds(base, K), :]
    return [v[VR * i:VR * (i + 1), :] for i in range(NV)]


def _store_list(ref, base, vs):
    ref[pl.ds(base, K), :] = jnp.concatenate(vs, axis=0)


def _merge_pair(mk, s_ref, i, desc, lanesplit=False):
    a = _load_list(s_ref, (2 * i) * K)
    b = _load_list(s_ref, (2 * i + 1) * K)
    t = [jnp.maximum(x, y) for x, y in zip(a, b)]
    if lanesplit:
        _refine_lanesplit(mk, t)
    else:
        _refine(mk, t, desc)
    _store_list(s_ref, i * K, t)


def _finalize(mk, s_ref, o_ref, n_chan):
    S = _load_list(s_ref, 0)
    lt = mk['lane_lt']
    t = []
    for v in S:
        pv = pltpu.roll(v, n_chan, axis=1)
        t.append(jnp.where(lt, jnp.maximum(v, pv), jnp.minimum(v, pv)))
    _refine(mk, t, desc=True)
    for i in range(NV):
        ssel = i // VR
        m = i % VR
        rows = [t[VR * m + d][ssel:ssel + 1, :n_chan] for d in range(VR)]
        o_ref[pl.ds(VR * i, VR), :] = jnp.concatenate(rows, axis=0)


def _topk_kernel(x_ref, o_ref, s_ref, *, n_blocks, n_chan):
    mk = _make_masks(n_chan)

    def sort_pair(p, _):
        for q in (0, 1):
            vs = _load_list(x_ref, (2 * p + q) * K)
            _sort_block(mk, vs, desc=(q == 0))
            _store_list(s_ref, (2 * p + q) * K, vs)
        return 0

    jax.lax.fori_loop(0, n_blocks // 2, sort_pair, 0)

    lists = n_blocks
    while lists > 2:
        m = lists // 2

        def merge_pair2(p, _):
            _merge_pair(mk, s_ref, 2 * p, desc=True)
            _merge_pair(mk, s_ref, 2 * p + 1, desc=False)
            return 0

        jax.lax.fori_loop(0, m // 2, merge_pair2, 0)
        lists = m

    _merge_pair(mk, s_ref, 0, desc=True, lanesplit=True)
    _finalize(mk, s_ref, o_ref, n_chan)


def kernel(x):
    B, L, C = x.shape
    rows = L // 2
    n_blocks = rows // K
    assert rows % K == 0 and n_blocks >= 2 and (n_blocks & (n_blocks - 1)) == 0
    assert C == 64
    xr = x.reshape(B * rows, 2 * C)

    body = functools.partial(_topk_kernel, n_blocks=n_blocks, n_chan=C)
    out = pl.pallas_call(
        body,
        grid=(B,),
        in_specs=[
            pl.BlockSpec((rows, 2 * C), lambda b: (b, 0)),
        ],
        out_specs=pl.BlockSpec((K, C), lambda b: (b, 0)),
        out_shape=jax.ShapeDtypeStruct((B * K, C), x.dtype),
        scratch_shapes=[pltpu.VMEM((rows, 2 * C), x.dtype)],
    )(xr)
    return out.reshape(B, K, C)
